# baseline (speedup 1.0000x reference)
import functools

import jax
import jax.numpy as jnp
from jax import lax
from jax.experimental import pallas as pl
from jax.experimental.pallas import tpu as pltpu
from jax.experimental.pallas import tpu_sc as plsc

N = 10000
E = 320000
D = 128
H = 128
OUT = 64

NC = 2
NS = 16
ROW = 128
EPW = 79 * ROW
EP = NC * NS * EPW
NROWS = EPW // ROW
NROWS0 = 91
NROWS1 = 2 * NROWS - NROWS0
RMAX = max(NROWS0, NROWS1)
ACC_ROWS = 10240
SLICE = ACC_ROWS // NS

_MESH = plsc.VectorSubcoreMesh(core_axis_name="c", subcore_axis_name="s")


def _sc_agg_body(with_deg, width, z_hbm, src_hbm, dst_hbm, zeros_hbm,
                 ones_hbm, zeros1_hbm, *refs):
  if with_deg:
    (part_hbm, deg_hbm, idx_src, idx_dst, zbuf, ones_v, acc_sh, deg_sh,
     *bufs) = refs
  else:
    part_hbm, idx_src, idx_dst, zbuf, acc_sh, *bufs = refs
    deg_hbm = ones_v = deg_sh = None

  cid = lax.axis_index("c")
  sid = lax.axis_index("s")
  w = cid * NS + sid

  pltpu.sync_copy(src_hbm.at[w], idx_src)
  pltpu.sync_copy(dst_hbm.at[w], idx_dst)
  pltpu.sync_copy(zeros_hbm, zbuf)
  if with_deg:
    pltpu.sync_copy(ones_hbm, ones_v)

  base = sid * SLICE
  off = 0
  while off < SLICE:
    sz = min(ROW, SLICE - off)
    pltpu.sync_copy(zbuf.at[pl.ds(0, sz)], acc_sh.at[pl.ds(base + off, sz)])
    off += sz
  if with_deg:
    pltpu.sync_copy(zeros1_hbm, deg_sh.at[pl.ds(base, SLICE)])
  plsc.subcore_barrier()

  def step(j, carry):
    pltpu.sync_copy(z_hbm.at[idx_src.at[j]], bufs[0])
    pltpu.sync_copy(bufs[0], acc_sh.at[idx_dst.at[j]], add=True)
    if with_deg:
      pltpu.sync_copy(ones_v, deg_sh.at[idx_dst.at[j]], add=True)
    return carry

  nrows = jnp.where(cid == 0, NROWS0, NROWS1)
  lax.fori_loop(0, nrows, step, 0)
  plsc.subcore_barrier()

  off = 0
  while off < SLICE:
    sz = min(ROW, SLICE - off)
    pltpu.sync_copy(acc_sh.at[pl.ds(base + off, sz)],
                    part_hbm.at[cid, pl.ds(base + off, sz)])
    off += sz
  if with_deg:
    pltpu.sync_copy(deg_sh.at[pl.ds(base, SLICE)],
                    deg_hbm.at[cid, 0, pl.ds(base, SLICE)])


def _sc_agg2_body(za_hbm, zb_hbm, src_hbm, dst_hbm, zeros_hbm, ones_hbm,
                  zeros1_hbm, parta_hbm, partb_hbm, deg_hbm,
                  idx_src, idx_dst, zbuf, ones_v, acc_sh, deg_sh, rowbuf):
  cid = lax.axis_index("c")
  sid = lax.axis_index("s")
  w = cid * NS + sid
  nrows = jnp.where(cid == 0, NROWS0, NROWS1)
  base = sid * SLICE

  pltpu.sync_copy(src_hbm.at[w], idx_src)
  pltpu.sync_copy(dst_hbm.at[w], idx_dst)
  pltpu.sync_copy(zeros_hbm, zbuf)
  pltpu.sync_copy(ones_hbm, ones_v)

  def zero_acc():
    off = 0
    while off < SLICE:
      sz = min(ROW, SLICE - off)
      pltpu.sync_copy(zbuf.at[pl.ds(0, sz)],
                      acc_sh.at[pl.ds(base + off, sz)])
      off += sz

  def writeout(part_hbm):
    off = 0
    while off < SLICE:
      sz = min(ROW, SLICE - off)
      pltpu.sync_copy(acc_sh.at[pl.ds(base + off, sz)],
                      part_hbm.at[cid, pl.ds(base + off, sz)])
      off += sz

  zero_acc()
  pltpu.sync_copy(zeros1_hbm, deg_sh.at[pl.ds(base, SLICE)])
  plsc.subcore_barrier()

  def step_a(j, carry):
    pltpu.sync_copy(za_hbm.at[idx_src.at[j]], rowbuf)
    pltpu.sync_copy(rowbuf, acc_sh.at[idx_dst.at[j]], add=True)
    pltpu.sync_copy(ones_v, deg_sh.at[idx_dst.at[j]], add=True)
    return carry

  lax.fori_loop(0, nrows, step_a, 0)
  plsc.subcore_barrier()
  writeout(parta_hbm)
  pltpu.sync_copy(deg_sh.at[pl.ds(base, SLICE)],
                  deg_hbm.at[cid, 0, pl.ds(base, SLICE)])
  zero_acc()
  plsc.subcore_barrier()

  def step_b(j, carry):
    pltpu.sync_copy(zb_hbm.at[idx_src.at[j]], rowbuf)
    pltpu.sync_copy(rowbuf, acc_sh.at[idx_dst.at[j]], add=True)
    return carry

  lax.fori_loop(0, nrows, step_b, 0)
  plsc.subcore_barrier()
  writeout(partb_hbm)


def _make_sc_agg2():
  out_type = [jax.ShapeDtypeStruct((NC, ACC_ROWS, OUT), jnp.float32),
              jax.ShapeDtypeStruct((NC, ACC_ROWS, OUT), jnp.float32),
              jax.ShapeDtypeStruct((NC, 1, ACC_ROWS), jnp.float32)]
  scratch = [
      pltpu.VMEM((RMAX, ROW), jnp.int32),
      pltpu.VMEM((RMAX, ROW), jnp.int32),
      pltpu.VMEM((ROW, OUT), jnp.float32),
      pltpu.VMEM((ROW,), jnp.float32),
      pltpu.VMEM_SHARED((ACC_ROWS, OUT), jnp.float32),
      pltpu.VMEM_SHARED((ACC_ROWS,), jnp.float32),
      pltpu.VMEM((ROW, OUT), jnp.float32),
  ]
  return pl.kernel(_sc_agg2_body, out_type=out_type, mesh=_MESH,
                   scratch_types=scratch,
                   compiler_params=pltpu.CompilerParams(
                       use_tc_tiling_on_sc=False))


def _make_sc_agg(width, with_deg):
  out_type = [jax.ShapeDtypeStruct((NC, ACC_ROWS, width), jnp.float32)]
  scratch = [
      pltpu.VMEM((RMAX, ROW), jnp.int32),
      pltpu.VMEM((RMAX, ROW), jnp.int32),
      pltpu.VMEM((ROW, width), jnp.float32),
  ]
  if with_deg:
    out_type.append(jax.ShapeDtypeStruct((NC, 1, ACC_ROWS), jnp.float32))
    scratch.append(pltpu.VMEM((ROW,), jnp.float32))
  scratch.append(pltpu.VMEM_SHARED((ACC_ROWS, width), jnp.float32))
  if with_deg:
    scratch.append(pltpu.VMEM_SHARED((ACC_ROWS,), jnp.float32))
  scratch.append(pltpu.VMEM((ROW, width), jnp.float32))

  body = functools.partial(_sc_agg_body, with_deg, width)
  return pl.kernel(body, out_type=out_type, mesh=_MESH,
                   scratch_types=scratch,
                   compiler_params=pltpu.CompilerParams(
                       use_tc_tiling_on_sc=False))


def _proj1_body(x_ref, wn_ref, ws_ref, bn_ref, bs_ref, za_ref, zb_ref,
                s_ref):
  x = x_ref[:]
  z = jnp.dot(x, wn_ref[:], preferred_element_type=jnp.float32)
  za_ref[:] = z[:, :OUT]
  zb_ref[:] = z[:, OUT:]
  s_ref[:] = (jnp.dot(x, ws_ref[:], preferred_element_type=jnp.float32)
              + bn_ref[:] + bs_ref[:])


def _layer1_body(p0a_ref, p1a_ref, p0b_ref, p1b_ref, d0_ref, d1_ref, s1_ref,
                 wn_ref, ws_ref, wo_ref, bn_ref, bs_ref, bo_ref,
                 z2_ref, s2_ref):
  deg = d0_ref[:] + d1_ref[:]
  rdeg = 1.0 / jnp.maximum(deg, 1.0)
  agg = jnp.concatenate(
      [(p0a_ref[:] + p1a_ref[:]) * rdeg, (p0b_ref[:] + p1b_ref[:]) * rdeg],
      axis=1)
  h1 = jnp.maximum(agg + s1_ref[:], 0.0)
  wo = wo_ref[:]
  wno = jnp.dot(wn_ref[:], wo, preferred_element_type=jnp.float32)
  wso = jnp.dot(ws_ref[:], wo, preferred_element_type=jnp.float32)
  b2 = (jnp.dot(bn_ref[:] + bs_ref[:], wo,
                preferred_element_type=jnp.float32) + bo_ref[:])
  z2_ref[:] = jnp.dot(h1, wno, preferred_element_type=jnp.float32)
  s2_ref[:] = (jnp.dot(h1, wso, preferred_element_type=jnp.float32) + b2)


def _final_body(q0_ref, q1_ref, d0_ref, d1_ref, s2_ref, out_ref):
  deg = d0_ref[:] + d1_ref[:]
  rdeg = 1.0 / jnp.maximum(deg, 1.0)
  out_ref[:] = (q0_ref[:] + q1_ref[:]) * rdeg + s2_ref[:]


_BM = 400
_GRID = (N // _BM,)


def _row_spec(w):
  return pl.BlockSpec((_BM, w), lambda i: (i, 0))


def _full_spec(r, c):
  return pl.BlockSpec((r, c), lambda i: (0, 0))


def kernel(features, edge_index, W_neigh, b_neigh, W_self, b_self,
           W_out, b_out):
  src = edge_index[0].astype(jnp.int32)
  dst = edge_index[1].astype(jnp.int32)
  pad = EP - E
  src_p = jnp.concatenate([src, jnp.zeros((pad,), jnp.int32)])
  dst_p = jnp.concatenate([dst, jnp.full((pad,), N, jnp.int32)])

  def to_tiles(flat, fill):
    rows = flat.reshape(EP // ROW, ROW)
    n0 = NS * NROWS0
    t0 = rows[:n0].reshape(NS, NROWS0, ROW)
    t0 = jnp.concatenate(
        [t0, jnp.full((NS, RMAX - NROWS0, ROW), fill, jnp.int32)], axis=1)
    t1 = rows[n0:].reshape(NS, NROWS1, ROW)
    if RMAX > NROWS1:
      t1 = jnp.concatenate(
          [t1, jnp.full((NS, RMAX - NROWS1, ROW), fill, jnp.int32)], axis=1)
    return jnp.concatenate([t0, t1], axis=0)

  src2d = to_tiles(src_p, 0)
  dst2d = to_tiles(dst_p, N)
  zeros128 = jnp.zeros((ROW, D), jnp.float32)
  zeros64 = jnp.zeros((ROW, OUT), jnp.float32)
  ones1 = jnp.ones((ROW,), jnp.float32)
  zeros1 = jnp.zeros((SLICE,), jnp.float32)
  bn = b_neigh.reshape(1, H)
  bs = b_self.reshape(1, H)
  bo = b_out.reshape(1, OUT)

  z1a, z1b, s1 = pl.pallas_call(
      _proj1_body,
      grid=_GRID,
      in_specs=[_row_spec(D), _full_spec(D, H), _full_spec(D, H),
                _full_spec(1, H), _full_spec(1, H)],
      out_specs=[_row_spec(OUT), _row_spec(OUT), _row_spec(H)],
      out_shape=[jax.ShapeDtypeStruct((N, OUT), jnp.float32),
                 jax.ShapeDtypeStruct((N, OUT), jnp.float32),
                 jax.ShapeDtypeStruct((N, H), jnp.float32)],
  )(features, W_neigh, W_self, bn, bs)

  part1a, part1b, degp = _make_sc_agg2()(z1a, z1b, src2d, dst2d, zeros64,
                                         ones1, zeros1)
  d0 = degp[0, 0, :N].reshape(N, 1)
  d1 = degp[1, 0, :N].reshape(N, 1)

  z2, s2 = pl.pallas_call(
      _layer1_body,
      grid=_GRID,
      in_specs=[_row_spec(OUT), _row_spec(OUT), _row_spec(OUT),
                _row_spec(OUT), _row_spec(1), _row_spec(1), _row_spec(H),
                _full_spec(D, H), _full_spec(D, H), _full_spec(H, OUT),
                _full_spec(1, H), _full_spec(1, H), _full_spec(1, OUT)],
      out_specs=[_row_spec(OUT), _row_spec(OUT)],
      out_shape=[jax.ShapeDtypeStruct((N, OUT), jnp.float32),
                 jax.ShapeDtypeStruct((N, OUT), jnp.float32)],
  )(part1a[0], part1a[1], part1b[0], part1b[1], d0, d1, s1,
    W_neigh, W_self, W_out, bn, bs, bo)

  part2 = _make_sc_agg(OUT, False)(z2, src2d, dst2d, zeros64, ones1,
                                   zeros1)[0]

  out = pl.pallas_call(
      _final_body,
      grid=_GRID,
      in_specs=[_row_spec(OUT), _row_spec(OUT), _row_spec(1), _row_spec(1),
                _row_spec(OUT)],
      out_specs=_row_spec(OUT),
      out_shape=jax.ShapeDtypeStruct((N, OUT), jnp.float32),
  )(part2[0], part2[1], d0, d1, s2)
  return out


# speedup vs baseline: 1.0337x; 1.0337x over previous
import functools

import jax
import jax.numpy as jnp
from jax import lax
from jax.experimental import pallas as pl
from jax.experimental.pallas import tpu as pltpu
from jax.experimental.pallas import tpu_sc as plsc

N = 10000
E = 320000
D = 128
H = 128
OUT = 64

NC = 2
NS = 16
ROW = 128
EPW = 79 * ROW
EP = NC * NS * EPW
NROWS = EPW // ROW
NROWS0 = 97
NROWS1 = 2 * NROWS - NROWS0
RMAX = max(NROWS0, NROWS1)
ACC_ROWS = 10240
SLICE = ACC_ROWS // NS

_MESH = plsc.VectorSubcoreMesh(core_axis_name="c", subcore_axis_name="s")


def _sc_agg_body(with_deg, width, z_hbm, src_hbm, dst_hbm, zeros_hbm,
                 ones_hbm, zeros1_hbm, *refs):
  if with_deg:
    (part_hbm, deg_hbm, idx_src, idx_dst, zbuf, ones_v, acc_sh, deg_sh,
     *bufs) = refs
  else:
    part_hbm, idx_src, idx_dst, zbuf, acc_sh, *bufs = refs
    deg_hbm = ones_v = deg_sh = None

  cid = lax.axis_index("c")
  sid = lax.axis_index("s")
  w = cid * NS + sid

  pltpu.sync_copy(src_hbm.at[w], idx_src)
  pltpu.sync_copy(dst_hbm.at[w], idx_dst)
  pltpu.sync_copy(zeros_hbm, zbuf)
  if with_deg:
    pltpu.sync_copy(ones_hbm, ones_v)

  base = sid * SLICE
  off = 0
  while off < SLICE:
    sz = min(ROW, SLICE - off)
    pltpu.sync_copy(zbuf.at[pl.ds(0, sz)], acc_sh.at[pl.ds(base + off, sz)])
    off += sz
  if with_deg:
    pltpu.sync_copy(zeros1_hbm, deg_sh.at[pl.ds(base, SLICE)])
  plsc.subcore_barrier()

  def step(j, carry):
    pltpu.sync_copy(z_hbm.at[idx_src.at[j]], bufs[0])
    pltpu.sync_copy(bufs[0], acc_sh.at[idx_dst.at[j]], add=True)
    if with_deg:
      pltpu.sync_copy(ones_v, deg_sh.at[idx_dst.at[j]], add=True)
    return carry

  nrows = jnp.where(cid == 0, NROWS0, NROWS1)
  lax.fori_loop(0, nrows, step, 0)
  plsc.subcore_barrier()

  off = 0
  while off < SLICE:
    sz = min(ROW, SLICE - off)
    pltpu.sync_copy(acc_sh.at[pl.ds(base + off, sz)],
                    part_hbm.at[cid, pl.ds(base + off, sz)])
    off += sz
  if with_deg:
    pltpu.sync_copy(deg_sh.at[pl.ds(base, SLICE)],
                    deg_hbm.at[cid, 0, pl.ds(base, SLICE)])


def _sc_agg2_body(za_hbm, zb_hbm, src_hbm, dst_hbm, zeros_hbm, ones_hbm,
                  zeros1_hbm, parta_hbm, partb_hbm, deg_hbm,
                  idx_src, idx_dst, zbuf, ones_v, acc_sh, deg_sh, rowbuf):
  cid = lax.axis_index("c")
  sid = lax.axis_index("s")
  w = cid * NS + sid
  nrows = jnp.where(cid == 0, NROWS0, NROWS1)
  base = sid * SLICE

  pltpu.sync_copy(src_hbm.at[w], idx_src)
  pltpu.sync_copy(dst_hbm.at[w], idx_dst)
  pltpu.sync_copy(zeros_hbm, zbuf)
  pltpu.sync_copy(ones_hbm, ones_v)

  def zero_acc():
    off = 0
    while off < SLICE:
      sz = min(ROW, SLICE - off)
      pltpu.sync_copy(zbuf.at[pl.ds(0, sz)],
                      acc_sh.at[pl.ds(base + off, sz)])
      off += sz

  def writeout(part_hbm):
    off = 0
    while off < SLICE:
      sz = min(ROW, SLICE - off)
      pltpu.sync_copy(acc_sh.at[pl.ds(base + off, sz)],
                      part_hbm.at[cid, pl.ds(base + off, sz)])
      off += sz

  zero_acc()
  pltpu.sync_copy(zeros1_hbm, deg_sh.at[pl.ds(base, SLICE)])
  plsc.subcore_barrier()

  def step_a(j, carry):
    pltpu.sync_copy(za_hbm.at[idx_src.at[j]], rowbuf)
    pltpu.sync_copy(rowbuf, acc_sh.at[idx_dst.at[j]], add=True)
    pltpu.sync_copy(ones_v, deg_sh.at[idx_dst.at[j]], add=True)
    return carry

  lax.fori_loop(0, nrows, step_a, 0)
  plsc.subcore_barrier()
  writeout(parta_hbm)
  pltpu.sync_copy(deg_sh.at[pl.ds(base, SLICE)],
                  deg_hbm.at[cid, 0, pl.ds(base, SLICE)])
  zero_acc()
  plsc.subcore_barrier()

  def step_b(j, carry):
    pltpu.sync_copy(zb_hbm.at[idx_src.at[j]], rowbuf)
    pltpu.sync_copy(rowbuf, acc_sh.at[idx_dst.at[j]], add=True)
    return carry

  lax.fori_loop(0, nrows, step_b, 0)
  plsc.subcore_barrier()
  writeout(partb_hbm)


def _make_sc_agg2():
  out_type = [jax.ShapeDtypeStruct((NC, ACC_ROWS, OUT), jnp.float32),
              jax.ShapeDtypeStruct((NC, ACC_ROWS, OUT), jnp.float32),
              jax.ShapeDtypeStruct((NC, 1, ACC_ROWS), jnp.float32)]
  scratch = [
      pltpu.VMEM((RMAX, ROW), jnp.int32),
      pltpu.VMEM((RMAX, ROW), jnp.int32),
      pltpu.VMEM((ROW, OUT), jnp.float32),
      pltpu.VMEM((ROW,), jnp.float32),
      pltpu.VMEM_SHARED((ACC_ROWS, OUT), jnp.float32),
      pltpu.VMEM_SHARED((ACC_ROWS,), jnp.float32),
      pltpu.VMEM((ROW, OUT), jnp.float32),
  ]
  return pl.kernel(_sc_agg2_body, out_type=out_type, mesh=_MESH,
                   scratch_types=scratch,
                   compiler_params=pltpu.CompilerParams(
                       use_tc_tiling_on_sc=False))


def _make_sc_agg(width, with_deg):
  out_type = [jax.ShapeDtypeStruct((NC, ACC_ROWS, width), jnp.float32)]
  scratch = [
      pltpu.VMEM((RMAX, ROW), jnp.int32),
      pltpu.VMEM((RMAX, ROW), jnp.int32),
      pltpu.VMEM((ROW, width), jnp.float32),
  ]
  if with_deg:
    out_type.append(jax.ShapeDtypeStruct((NC, 1, ACC_ROWS), jnp.float32))
    scratch.append(pltpu.VMEM((ROW,), jnp.float32))
  scratch.append(pltpu.VMEM_SHARED((ACC_ROWS, width), jnp.float32))
  if with_deg:
    scratch.append(pltpu.VMEM_SHARED((ACC_ROWS,), jnp.float32))
  scratch.append(pltpu.VMEM((ROW, width), jnp.float32))

  body = functools.partial(_sc_agg_body, with_deg, width)
  return pl.kernel(body, out_type=out_type, mesh=_MESH,
                   scratch_types=scratch,
                   compiler_params=pltpu.CompilerParams(
                       use_tc_tiling_on_sc=False))


def _proj1_body(x_ref, wn_ref, ws_ref, bn_ref, bs_ref, za_ref, zb_ref,
                s_ref):
  x = x_ref[:]
  z = jnp.dot(x, wn_ref[:], preferred_element_type=jnp.float32)
  za_ref[:] = z[:, :OUT]
  zb_ref[:] = z[:, OUT:]
  s_ref[:] = (jnp.dot(x, ws_ref[:], preferred_element_type=jnp.float32)
              + bn_ref[:] + bs_ref[:])


def _layer1_body(p0a_ref, p1a_ref, p0b_ref, p1b_ref, d0_ref, d1_ref, s1_ref,
                 wn_ref, ws_ref, wo_ref, bn_ref, bs_ref, bo_ref,
                 z2_ref, s2_ref):
  deg = d0_ref[:] + d1_ref[:]
  rdeg = 1.0 / jnp.maximum(deg, 1.0)
  agg = jnp.concatenate(
      [(p0a_ref[:] + p1a_ref[:]) * rdeg, (p0b_ref[:] + p1b_ref[:]) * rdeg],
      axis=1)
  h1 = jnp.maximum(agg + s1_ref[:], 0.0)
  wo = wo_ref[:]
  wno = jnp.dot(wn_ref[:], wo, preferred_element_type=jnp.float32)
  wso = jnp.dot(ws_ref[:], wo, preferred_element_type=jnp.float32)
  b2 = (jnp.dot(bn_ref[:] + bs_ref[:], wo,
                preferred_element_type=jnp.float32) + bo_ref[:])
  z2_ref[:] = jnp.dot(h1, wno, preferred_element_type=jnp.float32)
  s2_ref[:] = (jnp.dot(h1, wso, preferred_element_type=jnp.float32) + b2)


def _final_body(q0_ref, q1_ref, d0_ref, d1_ref, s2_ref, out_ref):
  deg = d0_ref[:] + d1_ref[:]
  rdeg = 1.0 / jnp.maximum(deg, 1.0)
  out_ref[:] = (q0_ref[:] + q1_ref[:]) * rdeg + s2_ref[:]


_BM = 400
_GRID = (N // _BM,)


def _row_spec(w):
  return pl.BlockSpec((_BM, w), lambda i: (i, 0))


def _full_spec(r, c):
  return pl.BlockSpec((r, c), lambda i: (0, 0))


def kernel(features, edge_index, W_neigh, b_neigh, W_self, b_self,
           W_out, b_out):
  src = edge_index[0].astype(jnp.int32)
  dst = edge_index[1].astype(jnp.int32)
  pad = EP - E
  src_p = jnp.concatenate([src, jnp.zeros((pad,), jnp.int32)])
  dst_p = jnp.concatenate([dst, jnp.full((pad,), N, jnp.int32)])

  def to_tiles(flat, fill):
    rows = flat.reshape(EP // ROW, ROW)
    n0 = NS * NROWS0
    t0 = rows[:n0].reshape(NS, NROWS0, ROW)
    t0 = jnp.concatenate(
        [t0, jnp.full((NS, RMAX - NROWS0, ROW), fill, jnp.int32)], axis=1)
    t1 = rows[n0:].reshape(NS, NROWS1, ROW)
    if RMAX > NROWS1:
      t1 = jnp.concatenate(
          [t1, jnp.full((NS, RMAX - NROWS1, ROW), fill, jnp.int32)], axis=1)
    return jnp.concatenate([t0, t1], axis=0)

  src2d = to_tiles(src_p, 0)
  dst2d = to_tiles(dst_p, N)
  zeros128 = jnp.zeros((ROW, D), jnp.float32)
  zeros64 = jnp.zeros((ROW, OUT), jnp.float32)
  ones1 = jnp.ones((ROW,), jnp.float32)
  zeros1 = jnp.zeros((SLICE,), jnp.float32)
  bn = b_neigh.reshape(1, H)
  bs = b_self.reshape(1, H)
  bo = b_out.reshape(1, OUT)

  z1a, z1b, s1 = pl.pallas_call(
      _proj1_body,
      grid=_GRID,
      in_specs=[_row_spec(D), _full_spec(D, H), _full_spec(D, H),
                _full_spec(1, H), _full_spec(1, H)],
      out_specs=[_row_spec(OUT), _row_spec(OUT), _row_spec(H)],
      out_shape=[jax.ShapeDtypeStruct((N, OUT), jnp.float32),
                 jax.ShapeDtypeStruct((N, OUT), jnp.float32),
                 jax.ShapeDtypeStruct((N, H), jnp.float32)],
  )(features, W_neigh, W_self, bn, bs)

  part1a, part1b, degp = _make_sc_agg2()(z1a, z1b, src2d, dst2d, zeros64,
                                         ones1, zeros1)
  d0 = degp[0, 0, :N].reshape(N, 1)
  d1 = degp[1, 0, :N].reshape(N, 1)

  z2, s2 = pl.pallas_call(
      _layer1_body,
      grid=_GRID,
      in_specs=[_row_spec(OUT), _row_spec(OUT), _row_spec(OUT),
                _row_spec(OUT), _row_spec(1), _row_spec(1), _row_spec(H),
                _full_spec(D, H), _full_spec(D, H), _full_spec(H, OUT),
                _full_spec(1, H), _full_spec(1, H), _full_spec(1, OUT)],
      out_specs=[_row_spec(OUT), _row_spec(OUT)],
      out_shape=[jax.ShapeDtypeStruct((N, OUT), jnp.float32),
                 jax.ShapeDtypeStruct((N, OUT), jnp.float32)],
  )(part1a[0], part1a[1], part1b[0], part1b[1], d0, d1, s1,
    W_neigh, W_self, W_out, bn, bs, bo)

  part2 = _make_sc_agg(OUT, False)(z2, src2d, dst2d, zeros64, ones1,
                                   zeros1)[0]

  out = pl.pallas_call(
      _final_body,
      grid=_GRID,
      in_specs=[_row_spec(OUT), _row_spec(OUT), _row_spec(1), _row_spec(1),
                _row_spec(OUT)],
      out_specs=_row_spec(OUT),
      out_shape=jax.ShapeDtypeStruct((N, OUT), jnp.float32),
  )(part2[0], part2[1], d0, d1, s2)
  return out
